# Initial kernel scaffold; baseline (speedup 1.0000x reference)
import jax
import jax.numpy as jnp
from jax.experimental import pallas as pl


def kernel(species, coordinates):
    raise NotImplementedError("write your pallas kernel here")


# speedup vs baseline: 2.2118x; 2.2118x over previous
import functools
import math

import jax
import jax.numpy as jnp
import numpy as np
from jax import lax
from jax.experimental import pallas as pl
from jax.experimental.pallas import tpu as pltpu
from jax.experimental.pallas import tpu_sc as plsc

M = 64
A = 24
NPAIR = A * A
NCH = 384
OUT_W = A * NCH

NC, NS, L = 2, 16, 16
NW = NC * NS

_RCR = 5.2
_RCA = 3.5
_ETAR = 16.0
_ETAA = 8.0
_SHFR = [0.9, 1.16875, 1.4375, 1.70625, 1.975, 2.24375, 2.5125, 2.78125,
         3.05, 3.31875, 3.5875, 3.85625, 4.125, 4.39375, 4.6625, 4.93125]
_SHFA = [0.9, 1.55, 2.2, 2.85]
_SHFZ = [(2 * k + 1) * math.pi / 16.0 for k in range(8)]
_COSZ = [math.cos(z) for z in _SHFZ]
_SINZ = [math.sin(z) for z in _SHFZ]

_CPOLY = [0.99999999228596, -4.934801387623153, 4.058698250549149,
          -1.3351743915873315, 0.23506322961458181, -0.0253909641009894,
          0.001605306471105794]

_jl, _kl = np.triu_indices(A, 1)
NPJK = len(_jl)
NPJK_PAD = ((NPJK + L - 1) // L) * L
_JKJ = np.zeros((NPJK_PAD,), np.int32)
_JKK = np.zeros((NPJK_PAD,), np.int32)
_JKJ[:NPJK] = _jl
_JKK[:NPJK] = _kl

_PT = np.zeros((4, 4), np.int32)
_c = 0
for _a in range(4):
    for _b in range(_a, 4):
        _PT[_a, _b] = _PT[_b, _a] = _c
        _c += 1
_PTAB = _PT.reshape(-1)


def _rsqrt(x, iters):
    i = plsc.bitcast(x, jnp.int32)
    i = jnp.int32(0x5F3759DF) - (i >> 1)
    y = plsc.bitcast(i, jnp.float32)
    for _ in range(iters):
        y = y * (jnp.float32(1.5) - jnp.float32(0.5) * x * y * y)
    return y


def _cos_pi(u):
    t = u * u
    acc = jnp.full((L,), _CPOLY[-1], jnp.float32)
    for c in _CPOLY[-2::-1]:
        acc = acc * t + jnp.float32(c)
    return acc


def _fc(d, cutoff):
    half = jnp.float32(0.5) * _cos_pi(d * jnp.float32(1.0 / cutoff))
    return jnp.where(d <= jnp.float32(cutoff),
                     half + jnp.float32(0.5), jnp.float32(0.0))


def _exp_g(arg):
    return jnp.exp(jnp.maximum(arg, jnp.float32(-100.0)))


def _aev_body(spec_hbm, coord_hbm, jkj_hbm, jkk_hbm, ptab_hbm, out_hbm,
              spec_v, coord_v, jkj_v, jkk_v, ptab_v,
              dist_v, fcr_v, fca_v, vx_v, vy_v, vz_v, acc_v):
    wid = lax.axis_index("s") * NC + lax.axis_index("c")

    pltpu.sync_copy(jkj_hbm, jkj_v)
    pltpu.sync_copy(jkk_hbm, jkk_v)
    pltpu.sync_copy(ptab_hbm, ptab_v)

    iota = lax.iota(jnp.int32, L)

    for mm in range(M // NW):
        m = wid * (M // NW) + mm
        pltpu.sync_copy(spec_hbm.at[pl.ds(m * A, A)], spec_v)
        pltpu.sync_copy(coord_hbm.at[pl.ds(m * 3 * A, 3 * A)], coord_v)

        def zero_body(c, carry):
            acc_v[pl.ds(c * L, L)] = jnp.zeros((L,), jnp.float32)
            return carry
        lax.fori_loop(0, OUT_W // L, zero_body, 0)

        def pair_body(c, carry):
            q = iota + c * L
            i = (q * 2731) >> 16
            j = q - i * A
            i3 = i * 3
            j3 = j * 3
            xi = plsc.load_gather(coord_v, [i3])
            yi = plsc.load_gather(coord_v, [i3 + 1])
            zi = plsc.load_gather(coord_v, [i3 + 2])
            xj = plsc.load_gather(coord_v, [j3])
            yj = plsc.load_gather(coord_v, [j3 + 1])
            zj = plsc.load_gather(coord_v, [j3 + 2])
            dx = xj - xi
            dy = yj - yi
            dz = zj - zi
            d2 = dx * dx + dy * dy + dz * dz + jnp.float32(1e-12)
            d = d2 * _rsqrt(d2, 3)
            d = jnp.where(i == j, jnp.float32(1e9), d)
            sl = pl.ds(c * L, L)
            dist_v[sl] = d
            fr = jnp.float32(0.25) * _fc(d, _RCR)
            fcr_v[sl] = fr
            fca_v[sl] = _fc(d, _RCA)
            vx_v[sl] = dx
            vy_v[sl] = dy
            vz_v[sl] = dz
            sj = plsc.load_gather(spec_v, [j])
            base = i * NCH + sj * 16
            for r in range(16):
                dr = d - jnp.float32(_SHFR[r])
                val = fr * _exp_g(jnp.float32(-_ETAR) * dr * dr)
                plsc.addupdate_scatter(acc_v, [base + r], val)
            return carry
        lax.fori_loop(0, NPAIR // L, pair_body, 0)

        def ang_chunk(i, t, carry):
            sl = pl.ds(t * L, L)
            jv = jkj_v[sl]
            kv = jkk_v[sl]
            ibase = i * A
            ij = ibase + jv
            ik = ibase + kv
            d1 = plsc.load_gather(dist_v, [ij])
            d2_ = plsc.load_gather(dist_v, [ik])
            fa = plsc.load_gather(fca_v, [ij]) * plsc.load_gather(fca_v, [ik])
            fa = jnp.where(jv < kv, fa, jnp.float32(0.0))
            inner = (plsc.load_gather(vx_v, [ij]) * plsc.load_gather(vx_v, [ik])
                     + plsc.load_gather(vy_v, [ij]) * plsc.load_gather(vy_v, [ik])
                     + plsc.load_gather(vz_v, [ij]) * plsc.load_gather(vz_v, [ik]))
            denom = jnp.maximum(d1 * d2_, jnp.float32(1e-8))
            ca = jnp.float32(0.95) * inner / denom
            ca = jnp.minimum(jnp.maximum(ca, jnp.float32(-0.95)), jnp.float32(0.95))
            s2 = jnp.float32(1.0) - ca * ca
            s = s2 * _rsqrt(s2, 2)
            dsum = (d1 + d2_) * jnp.float32(0.5)
            sj = plsc.load_gather(spec_v, [jv])
            sk = plsc.load_gather(spec_v, [kv])
            p = plsc.load_gather(ptab_v, [sj * 4 + sk])
            obase = p * 32 + (i * NCH + 64)
            f1s = []
            for zi in range(8):
                b = (jnp.float32(1.0) + ca * jnp.float32(_COSZ[zi])
                     + s * jnp.float32(_SINZ[zi])) * jnp.float32(0.5)
                for _ in range(5):
                    b = b * b
                f1s.append(b)
            fa2 = jnp.float32(2.0) * fa
            for ai in range(4):
                da = dsum - jnp.float32(_SHFA[ai])
                g = fa2 * _exp_g(jnp.float32(-_ETAA) * da * da)
                for zi in range(8):
                    plsc.addupdate_scatter(acc_v, [obase + (ai * 8 + zi)],
                                           g * f1s[zi])
            return carry

        def center_body(i, carry):
            return lax.fori_loop(
                0, NPJK_PAD // L,
                lambda t, cc: ang_chunk(i, t, cc), carry)
        lax.fori_loop(0, A, center_body, 0)

        pltpu.sync_copy(acc_v, out_hbm.at[m])


_mesh = plsc.VectorSubcoreMesh(core_axis_name="c", subcore_axis_name="s",
                               num_cores=NC, num_subcores=NS)

_aev_sc = functools.partial(
    pl.kernel,
    out_type=jax.ShapeDtypeStruct((M, OUT_W), jnp.float32),
    mesh=_mesh,
    compiler_params=pltpu.CompilerParams(needs_layout_passes=False),
    scratch_types=[
        pltpu.VMEM((A,), jnp.int32),
        pltpu.VMEM((3 * A,), jnp.float32),
        pltpu.VMEM((NPJK_PAD,), jnp.int32),
        pltpu.VMEM((NPJK_PAD,), jnp.int32),
        pltpu.VMEM((16,), jnp.int32),
        pltpu.VMEM((NPAIR,), jnp.float32),
        pltpu.VMEM((NPAIR,), jnp.float32),
        pltpu.VMEM((NPAIR,), jnp.float32),
        pltpu.VMEM((NPAIR,), jnp.float32),
        pltpu.VMEM((NPAIR,), jnp.float32),
        pltpu.VMEM((NPAIR,), jnp.float32),
        pltpu.VMEM((OUT_W,), jnp.float32),
    ],
)(_aev_body)


def kernel(species, coordinates):
    sp = species.reshape(-1).astype(jnp.int32)
    co = coordinates.reshape(-1).astype(jnp.float32)
    out = _aev_sc(sp, co, jnp.asarray(_JKJ), jnp.asarray(_JKK),
                  jnp.asarray(_PTAB))
    return out.reshape(M, A, NCH)
